# Initial kernel scaffold; baseline (speedup 1.0000x reference)
import jax
import jax.numpy as jnp
from jax.experimental import pallas as pl


def kernel(hidden_states, ln_w, gate_w, w13, w2, sg_w, su_w, sd_w):
    raise NotImplementedError("write your pallas kernel here")


# speedup vs baseline: 1.4327x; 1.4327x over previous
import functools

import jax
import jax.numpy as jnp
from jax.experimental import pallas as pl
from jax.experimental.pallas import tpu as pltpu

S = 2048
H = 2048
FF = 1024
E = 16
TM = 128
NT = 48
NSLOT = NT * TM
EPS = 1e-6
F32 = jnp.float32


def _fiota(shape, dim):
    return jax.lax.broadcasted_iota(jnp.int32, shape, dim).astype(F32)


def _route_kernel(hid_ref, lnw_ref, gwt_ref, xn_ref, st_ref, sw_ref, te_ref,
                  c_scr, cum_scr):
    x = hid_ref[:]
    v = jnp.mean(x * x, axis=1, keepdims=True)
    xn = (x * jax.lax.rsqrt(v + EPS)) * lnw_ref[:]
    xn_ref[:] = xn

    logits = jnp.dot(xn.astype(jnp.bfloat16), gwt_ref[:].astype(jnp.bfloat16),
                     preferred_element_type=F32)
    col = _fiota((S, E), 1)

    m1 = jnp.max(logits, axis=1, keepdims=True)
    i1 = jnp.min(jnp.where(logits == m1, col, float(E)), axis=1, keepdims=True)
    o1 = col == i1
    lm = jnp.where(o1, -jnp.inf, logits)
    m2 = jnp.max(lm, axis=1, keepdims=True)
    i2 = jnp.min(jnp.where(lm == m2, col, float(E)), axis=1, keepdims=True)
    o2 = col == i2

    e2 = jnp.exp(m2 - m1)
    wa = 1.0 / (1.0 + e2)
    wb = e2 * wa

    c_scr[:] = o1.astype(F32) + o2.astype(F32)
    CH = 128
    tri = (_fiota((CH, CH), 0)
           > _fiota((CH, CH), 1)).astype(F32)

    def chunk_body(c, carry):
        cc = c_scr[pl.ds(c * CH, CH), :]
        within = jnp.dot(tri, cc, preferred_element_type=F32,
                         precision=jax.lax.Precision.HIGHEST)
        cum_scr[pl.ds(c * CH, CH), :] = within + carry
        return carry + jnp.sum(cc, axis=0, keepdims=True)

    counts = jax.lax.fori_loop(0, S // CH, chunk_body,
                               jnp.zeros((1, E), F32))

    padded = jnp.floor((counts + (TM - 1)) / TM) * TM
    tcnt = padded / TM
    er = _fiota((E, E), 0)
    ec = _fiota((E, E), 1)
    pb = jnp.broadcast_to(padded, (E, E))
    off = jnp.sum(jnp.where(ec < er, pb, 0.0), axis=1)
    tb = jnp.broadcast_to(tcnt, (E, E))
    tile_end = jnp.sum(jnp.where(ec <= er, tb, 0.0), axis=1)

    jr = _fiota((NT, E), 0)
    te = jnp.sum((jr >= tile_end[None, :]).astype(F32), axis=1)
    te = jnp.minimum(te, float(E - 1))
    te_ref[:] = te[None, :].astype(jnp.int32)

    cum = cum_scr[:]
    offb = jnp.broadcast_to(off[None, :], (S, E))
    slot1 = (jnp.sum(jnp.where(o1, offb, 0.0), axis=1, keepdims=True)
             + jnp.sum(jnp.where(o1, cum, 0.0), axis=1, keepdims=True))
    slot2 = (jnp.sum(jnp.where(o2, offb, 0.0), axis=1, keepdims=True)
             + jnp.sum(jnp.where(o2, cum, 0.0), axis=1, keepdims=True))

    tokf = _fiota((S, 1), 0)
    r1 = jnp.concatenate([wa, tokf], axis=1)
    r2 = jnp.concatenate([wb, tokf], axis=1)
    dn = (((0,), (0,)), ((), ()))

    def slot_body(j, _):
        scol = _fiota((S, TM), 1) + (j * TM).astype(F32)
        a1 = (jnp.broadcast_to(slot1, (S, TM)) == scol).astype(F32)
        a2 = (jnp.broadcast_to(slot2, (S, TM)) == scol).astype(F32)
        hp = jax.lax.Precision.HIGHEST
        r = (jax.lax.dot_general(r1, a1, dn, preferred_element_type=F32,
                                 precision=hp)
             + jax.lax.dot_general(r2, a2, dn, preferred_element_type=F32,
                                   precision=hp))
        sw_ref[pl.ds(j, 1), :] = r[0:1, :]
        st_ref[pl.ds(j, 1), :] = r[1:2, :].astype(jnp.int32)
        return 0

    jax.lax.fori_loop(0, NT, slot_body, 0)


SUB = 8
NG = TM // SUB


def _moe_kernel(te_ref, st_ref, swf_ref, xn_ref, w13_ref, w2_ref, out_ref,
                xg_scr, y_scr):
    j = pl.program_id(0)

    @pl.when(j == 0)
    def _():
        out_ref[:] = jnp.zeros_like(out_ref)

    sub0 = jax.lax.broadcasted_iota(jnp.int32, (SUB, H), 0)

    def gbody(g, _):
        base = j * TM + g * SUB
        acc = jnp.zeros((SUB, H), F32)
        for k in range(SUB):
            t = st_ref[base + k]
            t8 = pl.multiple_of((t // SUB) * SUB, SUB)
            blk = xn_ref[pl.ds(t8, SUB), :]
            row = jnp.sum(jnp.where(sub0 == (t % SUB), blk, 0.0),
                          axis=0, keepdims=True)
            acc = acc + jnp.where(sub0 == k, row, 0.0)
        xg_scr[pl.ds(pl.multiple_of(g * SUB, SUB), SUB), :] = acc
        return 0

    jax.lax.fori_loop(0, NG, gbody, 0)

    gu = jnp.dot(xg_scr[:].astype(jnp.bfloat16), w13_ref[0],
                 preferred_element_type=F32)
    g = gu[:, :FF]
    u = gu[:, FF:]
    h = g * jax.lax.logistic(g) * u
    y_scr[:] = jnp.dot(h.astype(jnp.bfloat16), w2_ref[0],
                       preferred_element_type=F32)

    def sbody(g, _):
        base = j * TM + g * SUB
        y_blk = y_scr[pl.ds(pl.multiple_of(g * SUB, SUB), SUB), :]
        for k in range(SUB):
            t = st_ref[base + k]
            w = swf_ref[base + k]
            yrow = jnp.sum(jnp.where(sub0 == k, y_blk, 0.0),
                           axis=0, keepdims=True)
            t8 = pl.multiple_of((t // SUB) * SUB, SUB)
            cur = out_ref[pl.ds(t8, SUB), :]
            out_ref[pl.ds(t8, SUB), :] = cur + jnp.where(
                sub0 == (t % SUB), w * yrow, 0.0)
        return 0

    jax.lax.fori_loop(0, NG, sbody, 0)


def _shared_kernel(hid_ref, xn_ref, moe_ref, sgt_ref, sut_ref, sdt_ref,
                   out_ref):
    x = xn_ref[:].astype(jnp.bfloat16)
    g = jnp.dot(x, sgt_ref[:], preferred_element_type=F32)
    u = jnp.dot(x, sut_ref[:], preferred_element_type=F32)
    h = g * jax.lax.logistic(g) * u
    sh = jnp.dot(h.astype(jnp.bfloat16), sdt_ref[:],
                 preferred_element_type=F32)
    out_ref[:] = hid_ref[:] + moe_ref[:] + sh


def kernel(hidden_states, ln_w, gate_w, w13, w2, sg_w, su_w, sd_w):
    Bb, Ss, Hh = hidden_states.shape
    hid = hidden_states.reshape(Ss, Hh)

    xn, slot_tok, slot_wgt, tile_exp = pl.pallas_call(
        _route_kernel,
        out_shape=[
            jax.ShapeDtypeStruct((S, H), F32),
            jax.ShapeDtypeStruct((NT, TM), jnp.int32),
            jax.ShapeDtypeStruct((NT, TM), F32),
            jax.ShapeDtypeStruct((1, NT), jnp.int32),
        ],
        scratch_shapes=[pltpu.VMEM((S, E), F32), pltpu.VMEM((S, E), F32)],
    )(hid, ln_w.reshape(1, H), gate_w.T)

    grid_spec = pltpu.PrefetchScalarGridSpec(
        num_scalar_prefetch=3,
        grid=(NT,),
        in_specs=[
            pl.BlockSpec((S, H), lambda j, te, st, sw: (0, 0)),
            pl.BlockSpec((1, H, 2 * FF), lambda j, te, st, sw: (te[j], 0, 0)),
            pl.BlockSpec((1, FF, H), lambda j, te, st, sw: (te[j], 0, 0)),
        ],
        out_specs=pl.BlockSpec((S, H), lambda j, te, st, sw: (0, 0)),
        scratch_shapes=[pltpu.VMEM((TM, H), F32),
                        pltpu.VMEM((TM, H), F32)],
    )
    moe = pl.pallas_call(
        _moe_kernel,
        grid_spec=grid_spec,
        out_shape=jax.ShapeDtypeStruct((S, H), F32),
        compiler_params=pltpu.CompilerParams(
            dimension_semantics=("arbitrary",),
            vmem_limit_bytes=110 * 1024 * 1024,
        ),
    )(tile_exp.reshape(NT), slot_tok.reshape(NSLOT), slot_wgt.reshape(NSLOT),
      xn, w13.astype(jnp.bfloat16), w2.astype(jnp.bfloat16))

    RB = 256
    out = pl.pallas_call(
        _shared_kernel,
        grid=(S // RB,),
        in_specs=[
            pl.BlockSpec((RB, H), lambda i: (i, 0)),
            pl.BlockSpec((RB, H), lambda i: (i, 0)),
            pl.BlockSpec((RB, H), lambda i: (i, 0)),
            pl.BlockSpec((H, FF), lambda i: (0, 0)),
            pl.BlockSpec((H, FF), lambda i: (0, 0)),
            pl.BlockSpec((FF, H), lambda i: (0, 0)),
        ],
        out_specs=pl.BlockSpec((RB, H), lambda i: (i, 0)),
        out_shape=jax.ShapeDtypeStruct((S, H), F32),
        compiler_params=pltpu.CompilerParams(
            dimension_semantics=("arbitrary",),
        ),
    )(hid, xn, moe, sg_w.T.astype(jnp.bfloat16), su_w.T.astype(jnp.bfloat16),
      sd_w.T.astype(jnp.bfloat16))

    return out.reshape(Bb, Ss, Hh)
